# baseline (speedup 1.0000x reference)
import jax
import jax.numpy as jnp
from jax.experimental import pallas as pl
from jax.experimental.pallas import tpu as pltpu

SUB_LEVEL = 4
SUB_CENTERS = 1024
DIM = 256
BLOCK_N = 1024


def _vq_kernel(x_ref, cb_ref, qh_ref, qs_ref, hc0_ref, hc1_ref, hc2_ref,
               hc3_ref, stats_ref, h1_s, h2_s, h3_s, b2_s):
    hc_refs = (hc0_ref, hc1_ref, hc2_ref, hc3_ref)
    pid = pl.program_id(0)

    @pl.when(pid == 0)
    def _init():
        for k in range(8):
            stats_ref[k] = 0.0
        for level in range(SUB_LEVEL):
            c = cb_ref[level]
            h1 = c.astype(jnp.bfloat16)
            r1 = c - h1.astype(jnp.float32)
            h2 = r1.astype(jnp.bfloat16)
            h3 = (r1 - h2.astype(jnp.float32)).astype(jnp.bfloat16)
            h1_s[level] = h1
            h2_s[level] = h2
            h3_s[level] = h3
            b2_s[level] = 0.5 * jnp.sum(c * c, axis=1)

    NS = 4
    SB = BLOCK_N // NS
    lane_iota = jax.lax.broadcasted_iota(jnp.int32, (SB, SUB_CENTERS), 1)

    xs = [x_ref[pl.ds(s * SB, SB), :] for s in range(NS)]
    rs = list(xs)
    qss = [jnp.zeros_like(xs[s]) for s in range(NS)]
    sd_sum = 0.0
    hd_sum = 0.0
    hd_last = 0.0

    for level in range(SUB_LEVEL):
        c = cb_ref[level]
        b2h = b2_s[level][None, :]
        hd_level = 0.0
        for s in range(NS):
            x = xs[s]
            r = rs[s]
            a2h = 0.5 * jnp.sum(r * r, axis=1, keepdims=True)
            g = jax.lax.dot_general(
                r, c, (((1,), (1,)), ((), ())),
                preferred_element_type=jnp.float32)
            diffh = (a2h + b2h) - g
            maxih = jnp.max(diffh, axis=1, keepdims=True)
            q = diffh / maxih
            qmin = jnp.min(q, axis=1, keepdims=True)
            xarg = qmin - q
            e = 1.0 + xarg * (1.0 + xarg * (0.5 + xarg * (1.0 / 6.0)))
            soft = jax.lax.dot_general(
                e.astype(jnp.bfloat16), h1_s[level],
                (((1,), (0,)), ((), ())),
                preferred_element_type=jnp.float32)
            soft = soft * (1.0 / jnp.sum(e, axis=1, keepdims=True))
            code = jnp.min(jnp.where(q == qmin, lane_iota, SUB_CENTERS),
                           axis=1).astype(jnp.int32)
            onehot = (lane_iota == code[:, None]).astype(jnp.bfloat16)
            hard = jax.lax.dot_general(
                onehot, h1_s[level], (((1,), (0,)), ((), ())),
                preferred_element_type=jnp.float32)
            hard = hard + jax.lax.dot_general(
                onehot, h2_s[level], (((1,), (0,)), ((), ())),
                preferred_element_type=jnp.float32)
            hard = hard + jax.lax.dot_general(
                onehot, h3_s[level], (((1,), (0,)), ((), ())),
                preferred_element_type=jnp.float32)
            r = r - hard
            rs[s] = r
            qs = qss[s] = qss[s] + soft
            t = x - qs
            sd_sum = sd_sum + jnp.sum(t * t)
            hd_level = hd_level + jnp.sum(r * r)
            hc_refs[level][pl.ds(s * SB, SB), :] = code[:, None]
        hd_sum = hd_sum + hd_level
        if level == SUB_LEVEL - 1:
            hd_last = hd_level

    jc_sum = 0.0
    for s in range(NS):
        qh = xs[s] - rs[s]
        qh_ref[pl.ds(s * SB, SB), :] = qh
        qs_ref[pl.ds(s * SB, SB), :] = qss[s]
        d = qss[s] - qh
        jc_sum = jc_sum + jnp.sum(d * d)

    stats_ref[0] += sd_sum
    stats_ref[1] += hd_sum
    stats_ref[2] += hd_last
    stats_ref[3] += jc_sum


@jax.jit
def kernel(input, Codebook):
    n, d = input.shape
    num_blocks = n // BLOCK_N
    grid = (num_blocks,)
    out_shape = (
        jax.ShapeDtypeStruct((n, d), jnp.float32),
        jax.ShapeDtypeStruct((n, d), jnp.float32),
        jax.ShapeDtypeStruct((n, 1), jnp.int32),
        jax.ShapeDtypeStruct((n, 1), jnp.int32),
        jax.ShapeDtypeStruct((n, 1), jnp.int32),
        jax.ShapeDtypeStruct((n, 1), jnp.int32),
        jax.ShapeDtypeStruct((8,), jnp.float32),
    )
    row_spec = pl.BlockSpec((BLOCK_N, d), lambda i: (i, 0))
    code_spec = pl.BlockSpec((BLOCK_N, 1), lambda i: (i, 0))
    out = pl.pallas_call(
        _vq_kernel,
        grid=grid,
        in_specs=[
            row_spec,
            pl.BlockSpec((SUB_LEVEL, SUB_CENTERS, DIM), lambda i: (0, 0, 0)),
        ],
        out_specs=(
            row_spec,
            row_spec,
            code_spec,
            code_spec,
            code_spec,
            code_spec,
            pl.BlockSpec(memory_space=pltpu.SMEM),
        ),
        out_shape=out_shape,
        scratch_shapes=[
            pltpu.VMEM((SUB_LEVEL, SUB_CENTERS, DIM), jnp.bfloat16),
            pltpu.VMEM((SUB_LEVEL, SUB_CENTERS, DIM), jnp.bfloat16),
            pltpu.VMEM((SUB_LEVEL, SUB_CENTERS, DIM), jnp.bfloat16),
            pltpu.VMEM((SUB_LEVEL, SUB_CENTERS), jnp.float32),
        ],
    )(input, Codebook)
    qhard, qsoft, hc0, hc1, hc2, hc3, stats = out
    nf = jnp.float32(n)
    soft_distortion = stats[0] / nf
    hard_distortion = stats[1] / nf
    error = stats[2] / nf
    joint_center = stats[3] / (nf * jnp.float32(d))
    hard_code = jnp.concatenate([hc0, hc1, hc2, hc3], axis=1)
    return (qhard, qsoft, soft_distortion, hard_distortion, joint_center,
            error, hard_code)


# speedup vs baseline: 1.1046x; 1.1046x over previous
import jax
import jax.numpy as jnp
from jax.experimental import pallas as pl
from jax.experimental.pallas import tpu as pltpu

SUB_LEVEL = 4
SUB_CENTERS = 1024
DIM = 256
BLOCK_N = 1024


def _vq_kernel(x_ref, cb_ref, qh_ref, qs_ref, hc0_ref, hc1_ref, hc2_ref,
               hc3_ref, stats_ref, h1_s, h2_s, h3_s, b2_s):
    hc_refs = (hc0_ref, hc1_ref, hc2_ref, hc3_ref)
    pid = pl.program_id(0)

    @pl.when(pid == 0)
    def _init():
        for k in range(8):
            stats_ref[k] = 0.0
        for level in range(SUB_LEVEL):
            c = cb_ref[level]
            h1 = c.astype(jnp.bfloat16)
            r1 = c - h1.astype(jnp.float32)
            h2 = r1.astype(jnp.bfloat16)
            h3 = (r1 - h2.astype(jnp.float32)).astype(jnp.bfloat16)
            h1_s[level] = h1
            h2_s[level] = h2
            h3_s[level] = h3
            b2_s[level] = 0.5 * jnp.sum(c * c, axis=1)

    NS = 4
    SB = BLOCK_N // NS
    lane_iota = jax.lax.broadcasted_iota(jnp.int32, (SB, SUB_CENTERS), 1)

    xs = [x_ref[pl.ds(s * SB, SB), :] for s in range(NS)]
    rs = list(xs)
    qss = [jnp.zeros_like(xs[s]) for s in range(NS)]
    sd_sum = 0.0
    hd_sum = 0.0
    hd_last = 0.0

    for level in range(SUB_LEVEL):
        c = cb_ref[level]
        b2h = b2_s[level][None, :]
        hd_level = 0.0
        for s in range(NS):
            x = xs[s]
            r = rs[s]
            a2h = 0.5 * jnp.sum(r * r, axis=1, keepdims=True)
            g = jax.lax.dot_general(
                r, c, (((1,), (1,)), ((), ())),
                preferred_element_type=jnp.float32)
            diffh = (a2h + b2h) - g
            maxih = jnp.max(diffh, axis=1, keepdims=True)
            q = diffh / maxih
            qmin = jnp.min(q, axis=1, keepdims=True)
            e = jnp.exp(qmin - q)
            soft = jax.lax.dot_general(
                e, c, (((1,), (0,)), ((), ())),
                preferred_element_type=jnp.float32)
            soft = soft * (1.0 / jnp.sum(e, axis=1, keepdims=True))
            code = jnp.min(jnp.where(q == qmin, lane_iota, SUB_CENTERS),
                           axis=1).astype(jnp.int32)
            onehot = (lane_iota == code[:, None]).astype(jnp.bfloat16)
            hard = jax.lax.dot_general(
                onehot, h1_s[level], (((1,), (0,)), ((), ())),
                preferred_element_type=jnp.float32)
            hard = hard + jax.lax.dot_general(
                onehot, h2_s[level], (((1,), (0,)), ((), ())),
                preferred_element_type=jnp.float32)
            hard = hard + jax.lax.dot_general(
                onehot, h3_s[level], (((1,), (0,)), ((), ())),
                preferred_element_type=jnp.float32)
            r = r - hard
            rs[s] = r
            qs = qss[s] = qss[s] + soft
            t = x - qs
            sd_sum = sd_sum + jnp.sum(t * t)
            hd_level = hd_level + jnp.sum(r * r)
            hc_refs[level][pl.ds(s * SB, SB), :] = code[:, None]
        hd_sum = hd_sum + hd_level
        if level == SUB_LEVEL - 1:
            hd_last = hd_level

    jc_sum = 0.0
    for s in range(NS):
        qh = xs[s] - rs[s]
        qh_ref[pl.ds(s * SB, SB), :] = qh
        qs_ref[pl.ds(s * SB, SB), :] = qss[s]
        d = qss[s] - qh
        jc_sum = jc_sum + jnp.sum(d * d)

    stats_ref[0] += sd_sum
    stats_ref[1] += hd_sum
    stats_ref[2] += hd_last
    stats_ref[3] += jc_sum


@jax.jit
def kernel(input, Codebook):
    n, d = input.shape
    num_blocks = n // BLOCK_N
    grid = (num_blocks,)
    out_shape = (
        jax.ShapeDtypeStruct((n, d), jnp.float32),
        jax.ShapeDtypeStruct((n, d), jnp.float32),
        jax.ShapeDtypeStruct((n, 1), jnp.int32),
        jax.ShapeDtypeStruct((n, 1), jnp.int32),
        jax.ShapeDtypeStruct((n, 1), jnp.int32),
        jax.ShapeDtypeStruct((n, 1), jnp.int32),
        jax.ShapeDtypeStruct((8,), jnp.float32),
    )
    row_spec = pl.BlockSpec((BLOCK_N, d), lambda i: (i, 0))
    code_spec = pl.BlockSpec((BLOCK_N, 1), lambda i: (i, 0))
    out = pl.pallas_call(
        _vq_kernel,
        grid=grid,
        in_specs=[
            row_spec,
            pl.BlockSpec((SUB_LEVEL, SUB_CENTERS, DIM), lambda i: (0, 0, 0)),
        ],
        out_specs=(
            row_spec,
            row_spec,
            code_spec,
            code_spec,
            code_spec,
            code_spec,
            pl.BlockSpec(memory_space=pltpu.SMEM),
        ),
        out_shape=out_shape,
        scratch_shapes=[
            pltpu.VMEM((SUB_LEVEL, SUB_CENTERS, DIM), jnp.bfloat16),
            pltpu.VMEM((SUB_LEVEL, SUB_CENTERS, DIM), jnp.bfloat16),
            pltpu.VMEM((SUB_LEVEL, SUB_CENTERS, DIM), jnp.bfloat16),
            pltpu.VMEM((SUB_LEVEL, SUB_CENTERS), jnp.float32),
        ],
    )(input, Codebook)
    qhard, qsoft, hc0, hc1, hc2, hc3, stats = out
    nf = jnp.float32(n)
    soft_distortion = stats[0] / nf
    hard_distortion = stats[1] / nf
    error = stats[2] / nf
    joint_center = stats[3] / (nf * jnp.float32(d))
    hard_code = jnp.concatenate([hc0, hc1, hc2, hc3], axis=1)
    return (qhard, qsoft, soft_distortion, hard_distortion, joint_center,
            error, hard_code)


# speedup vs baseline: 1.1840x; 1.0719x over previous
import jax
import jax.numpy as jnp
from jax.experimental import pallas as pl
from jax.experimental.pallas import tpu as pltpu

SUB_LEVEL = 4
SUB_CENTERS = 1024
DIM = 256
BLOCK_N = 1024


def _vq_kernel(x_ref, cb_ref, qh_ref, qs_ref, hc0_ref, hc1_ref, hc2_ref,
               hc3_ref, stats_ref, h1_s, h2_s, h3_s, b2_s):
    hc_refs = (hc0_ref, hc1_ref, hc2_ref, hc3_ref)
    pid = pl.program_id(0)

    @pl.when(pid == 0)
    def _init():
        for k in range(8):
            stats_ref[k] = 0.0
        for level in range(SUB_LEVEL):
            c = cb_ref[level]
            h1 = c.astype(jnp.bfloat16)
            r1 = c - h1.astype(jnp.float32)
            h2 = r1.astype(jnp.bfloat16)
            h3 = (r1 - h2.astype(jnp.float32)).astype(jnp.bfloat16)
            h1_s[level] = h1
            h2_s[level] = h2
            h3_s[level] = h3
            b2_s[level] = 0.5 * jnp.sum(c * c, axis=1)

    NS = 2
    SB = BLOCK_N // NS
    lane_iota = jax.lax.broadcasted_iota(jnp.int32, (SB, SUB_CENTERS), 1)

    xs = [x_ref[pl.ds(s * SB, SB), :] for s in range(NS)]
    rs = list(xs)
    qss = [jnp.zeros_like(xs[s]) for s in range(NS)]
    sd_sum = 0.0
    hd_sum = 0.0
    hd_last = 0.0

    for level in range(SUB_LEVEL):
        c = cb_ref[level]
        b2h = b2_s[level][None, :]
        hd_level = 0.0
        for s in range(NS):
            x = xs[s]
            r = rs[s]
            a2h = 0.5 * jnp.sum(r * r, axis=1, keepdims=True)
            g = jax.lax.dot_general(
                r, c, (((1,), (1,)), ((), ())),
                preferred_element_type=jnp.float32)
            diffh = (a2h + b2h) - g
            maxih = jnp.max(diffh, axis=1, keepdims=True)
            q = diffh / maxih
            qmin = jnp.min(q, axis=1, keepdims=True)
            e = jnp.exp(qmin - q)
            soft = jax.lax.dot_general(
                e, c, (((1,), (0,)), ((), ())),
                preferred_element_type=jnp.float32)
            soft = soft * (1.0 / jnp.sum(e, axis=1, keepdims=True))
            code = jnp.min(jnp.where(q == qmin, lane_iota, SUB_CENTERS),
                           axis=1).astype(jnp.int32)
            onehot = (lane_iota == code[:, None]).astype(jnp.bfloat16)
            hard = jax.lax.dot_general(
                onehot, h1_s[level], (((1,), (0,)), ((), ())),
                preferred_element_type=jnp.float32)
            hard = hard + jax.lax.dot_general(
                onehot, h2_s[level], (((1,), (0,)), ((), ())),
                preferred_element_type=jnp.float32)
            hard = hard + jax.lax.dot_general(
                onehot, h3_s[level], (((1,), (0,)), ((), ())),
                preferred_element_type=jnp.float32)
            r = r - hard
            rs[s] = r
            qs = qss[s] = qss[s] + soft
            t = x - qs
            sd_sum = sd_sum + jnp.sum(t * t)
            hd_level = hd_level + jnp.sum(r * r)
            hc_refs[level][pl.ds(s * SB, SB), :] = code[:, None]
        hd_sum = hd_sum + hd_level
        if level == SUB_LEVEL - 1:
            hd_last = hd_level

    jc_sum = 0.0
    for s in range(NS):
        qh = xs[s] - rs[s]
        qh_ref[pl.ds(s * SB, SB), :] = qh
        qs_ref[pl.ds(s * SB, SB), :] = qss[s]
        d = qss[s] - qh
        jc_sum = jc_sum + jnp.sum(d * d)

    stats_ref[0] += sd_sum
    stats_ref[1] += hd_sum
    stats_ref[2] += hd_last
    stats_ref[3] += jc_sum


@jax.jit
def kernel(input, Codebook):
    n, d = input.shape
    num_blocks = n // BLOCK_N
    grid = (num_blocks,)
    out_shape = (
        jax.ShapeDtypeStruct((n, d), jnp.float32),
        jax.ShapeDtypeStruct((n, d), jnp.float32),
        jax.ShapeDtypeStruct((n, 1), jnp.int32),
        jax.ShapeDtypeStruct((n, 1), jnp.int32),
        jax.ShapeDtypeStruct((n, 1), jnp.int32),
        jax.ShapeDtypeStruct((n, 1), jnp.int32),
        jax.ShapeDtypeStruct((8,), jnp.float32),
    )
    row_spec = pl.BlockSpec((BLOCK_N, d), lambda i: (i, 0))
    code_spec = pl.BlockSpec((BLOCK_N, 1), lambda i: (i, 0))
    out = pl.pallas_call(
        _vq_kernel,
        grid=grid,
        in_specs=[
            row_spec,
            pl.BlockSpec((SUB_LEVEL, SUB_CENTERS, DIM), lambda i: (0, 0, 0)),
        ],
        out_specs=(
            row_spec,
            row_spec,
            code_spec,
            code_spec,
            code_spec,
            code_spec,
            pl.BlockSpec(memory_space=pltpu.SMEM),
        ),
        out_shape=out_shape,
        scratch_shapes=[
            pltpu.VMEM((SUB_LEVEL, SUB_CENTERS, DIM), jnp.bfloat16),
            pltpu.VMEM((SUB_LEVEL, SUB_CENTERS, DIM), jnp.bfloat16),
            pltpu.VMEM((SUB_LEVEL, SUB_CENTERS, DIM), jnp.bfloat16),
            pltpu.VMEM((SUB_LEVEL, SUB_CENTERS), jnp.float32),
        ],
    )(input, Codebook)
    qhard, qsoft, hc0, hc1, hc2, hc3, stats = out
    nf = jnp.float32(n)
    soft_distortion = stats[0] / nf
    hard_distortion = stats[1] / nf
    error = stats[2] / nf
    joint_center = stats[3] / (nf * jnp.float32(d))
    hard_code = jnp.concatenate([hc0, hc1, hc2, hc3], axis=1)
    return (qhard, qsoft, soft_distortion, hard_distortion, joint_center,
            error, hard_code)
